# Initial kernel scaffold; baseline (speedup 1.0000x reference)
import jax
import jax.numpy as jnp
from jax.experimental import pallas as pl


def kernel(flows, images, decay_log):
    raise NotImplementedError("write your pallas kernel here")


# speedup vs baseline: 6.0580x; 6.0580x over previous
import functools

import jax
import jax.numpy as jnp
from jax import lax
from jax.experimental import pallas as pl
from jax.experimental.pallas import tpu as pltpu
from jax.experimental.pallas import tpu_sc as plsc

WINDOW = 8
NC = 2
NS = 16
LANES = 16
P = 512
G = 128


def _splat(ref, i):
    return plsc.load_gather(ref, [jnp.full((LANES,), i, jnp.int32)])


def _make_sc_call(B, L, C, H, W, interpret=False):
    HW = H * W
    P = min(512, HW)
    G = min(128, P)
    mesh = plsc.VectorSubcoreMesh(core_axis_name="c", subcore_axis_name="s",
                                  num_cores=NC, num_subcores=NS)

    @functools.partial(
        pl.kernel,
        out_type=jax.ShapeDtypeStruct((B * L, HW, C), jnp.float32),
        mesh=mesh,
        compiler_params=pltpu.CompilerParams(
            needs_layout_passes=False, use_tc_tiling_on_sc=False),
        interpret=interpret,
        scratch_types=[
            pltpu.VMEM((P,), jnp.float32),
            pltpu.VMEM((P,), jnp.float32),
            pltpu.VMEM((P,), jnp.float32),
            pltpu.VMEM((P,), jnp.float32),
            pltpu.VMEM((4, P // G, G), jnp.int32),
            pltpu.VMEM((P,), jnp.float32),
            pltpu.VMEM((P,), jnp.float32),
            pltpu.VMEM((P,), jnp.float32),
            pltpu.VMEM((P,), jnp.float32),
            pltpu.VMEM((P, C), jnp.float32),
            pltpu.VMEM((P, C), jnp.float32),
            pltpu.VMEM((LANES,), jnp.float32),
            pltpu.SemaphoreType.DMA,
        ],
    )
    def sc_call(cpb_hbm, cum_hbm, img_hbm, wk_hbm, out_hbm,
                glx, gly, ckx, cky, idx4, w_a, w_b, w_c, w_d,
                rows, acc, wkv, sem):
        wbufs = (w_a, w_b, w_c, w_d)
        cid = lax.axis_index("c")
        sid = lax.axis_index("s")
        f = sid * NC + cid
        b = f // L
        l = f % L
        k0 = jnp.maximum(l - (WINDOW - 1), 0)
        pltpu.sync_copy(wk_hbm, wkv)

        def chunk_body(ci, _):
            c0 = pl.multiple_of(ci * P, P)
            pltpu.sync_copy(cpb_hbm.at[f, 0, pl.ds(c0, P)], glx)
            pltpu.sync_copy(cpb_hbm.at[f, 1, pl.ds(c0, P)], gly)

            def zero_body(p, _):
                z = jnp.zeros((LANES,), jnp.float32)
                acc[p, pl.ds(0, LANES)] = z
                acc[p, pl.ds(LANES, LANES)] = z
                return 0
            lax.fori_loop(0, P, zero_body, 0)

            def k_body(k, _):
                bk = b * L + k
                pltpu.sync_copy(cum_hbm.at[bk, 0, pl.ds(c0, P)], ckx)
                pltpu.sync_copy(cum_hbm.at[bk, 1, pl.ds(c0, P)], cky)
                wkd = _splat(wkv, l - k)
                rowbase = bk * HW

                for g in range(P // LANES):
                    s = pl.ds(g * LANES, LANES)
                    gx = glx[s] - ckx[s]
                    gy = gly[s] - cky[s]
                    ix = (gx + 1.0) * (W * 0.5) - 0.5
                    iy = (gy + 1.0) * (H * 0.5) - 0.5
                    xt = ix.astype(jnp.int32)
                    x0 = xt - (xt.astype(jnp.float32) > ix).astype(jnp.int32)
                    fx = ix - x0.astype(jnp.float32)
                    yt = iy.astype(jnp.int32)
                    y0 = yt - (yt.astype(jnp.float32) > iy).astype(jnp.int32)
                    fy = iy - y0.astype(jnp.float32)
                    x1 = x0 + 1
                    y1 = y0 + 1
                    vx0 = (x0 >= 0) & (x0 < W)
                    vx1 = (x1 >= 0) & (x1 < W)
                    vy0 = (y0 >= 0) & (y0 < H)
                    vy1 = (y1 >= 0) & (y1 < H)
                    cx0 = jnp.clip(x0, 0, W - 1)
                    cx1 = jnp.clip(x1, 0, W - 1)
                    ry0 = rowbase + jnp.clip(y0, 0, H - 1) * W
                    ry1 = rowbase + jnp.clip(y1, 0, H - 1) * W
                    ofx = 1.0 - fx
                    ofy = 1.0 - fy
                    zero = jnp.zeros((LANES,), jnp.float32)
                    wa = jnp.where(vx0 & vy0, wkd * (ofx * ofy), zero)
                    wb = jnp.where(vx0 & vy1, wkd * (ofx * fy), zero)
                    wc = jnp.where(vx1 & vy0, wkd * (fx * ofy), zero)
                    wd = jnp.where(vx1 & vy1, wkd * (fx * fy), zero)
                    j, o = divmod(g * LANES, G)
                    so = pl.ds(o, LANES)
                    idx4[0, j, so] = ry0 + cx0
                    idx4[1, j, so] = ry1 + cx0
                    idx4[2, j, so] = ry0 + cx1
                    idx4[3, j, so] = ry1 + cx1
                    w_a[s] = wa
                    w_b[s] = wb
                    w_c[s] = wc
                    w_d[s] = wd

                for t in range(4):
                    cps = [
                        pltpu.async_copy(
                            img_hbm.at[idx4.at[t, j]],
                            rows.at[pl.ds(j * G, G)],
                            sem,
                        )
                        for j in range(P // G)
                    ]
                    for cp in cps:
                        cp.wait()

                    def accum_body(p, _, t=t):
                        wv = _splat(wbufs[t], p)
                        plsc.addupdate(acc.at[p, pl.ds(0, LANES)],
                                       wv * rows[p, pl.ds(0, LANES)])
                        plsc.addupdate(acc.at[p, pl.ds(LANES, LANES)],
                                       wv * rows[p, pl.ds(LANES, LANES)])
                        return 0
                    lax.fori_loop(0, P, accum_body, 0)
                return 0

            lax.fori_loop(k0, l + 1, k_body, 0)
            pltpu.sync_copy(acc, out_hbm.at[f, pl.ds(c0, P), :])
            return 0

        lax.fori_loop(0, HW // P, chunk_body, 0)

    return sc_call


def kernel(flows, images, decay_log):
    B, L, C, H, W = images.shape
    HW = H * W
    cum = jnp.cumsum(flows.astype(jnp.float32), axis=1)
    gx = jnp.linspace(-1.0 + 1.0 / W, 1.0 - 1.0 / W, W)
    gy = jnp.linspace(-1.0 + 1.0 / H, 1.0 - 1.0 / H, H)
    mx, my = jnp.meshgrid(gx, gy, indexing="xy")
    base = jnp.stack([mx, my], axis=0).astype(jnp.float32)
    cpb = cum + base[None, None]
    cum2 = cum.reshape(B * L, 2, HW)
    cpb2 = cpb.reshape(B * L, 2, HW)
    imgflat = (images.astype(jnp.float32)
               .transpose(0, 1, 3, 4, 2)
               .reshape(B * L * HW, C))
    decay = jnp.exp(decay_log)
    dist = jnp.arange(LANES, dtype=jnp.float32)
    wks = jnp.exp(-decay * dist)
    out = _make_sc_call(B, L, C, H, W)(cpb2, cum2, imgflat, wks)
    out = out.reshape(B, L, H, W, C).transpose(0, 1, 4, 2, 3)
    return out.astype(images.dtype)


# speedup vs baseline: 7.0374x; 1.1617x over previous
import functools

import jax
import jax.numpy as jnp
from jax import lax
from jax.experimental import pallas as pl
from jax.experimental.pallas import tpu as pltpu
from jax.experimental.pallas import tpu_sc as plsc

WINDOW = 8
NC = 2
NS = 16
LANES = 16
P = 512
G = 128
UNROLL = 16


def _splat(ref, i):
    return plsc.load_gather(ref, [jnp.full((LANES,), i, jnp.int32)])


def _make_sc_call(B, L, C, H, W, interpret=False):
    HW = H * W
    P = min(512, HW)
    G = min(128, P)
    mesh = plsc.VectorSubcoreMesh(core_axis_name="c", subcore_axis_name="s",
                                  num_cores=NC, num_subcores=NS)

    @functools.partial(
        pl.kernel,
        out_type=jax.ShapeDtypeStruct((B * L, HW, C), jnp.float32),
        mesh=mesh,
        compiler_params=pltpu.CompilerParams(
            needs_layout_passes=False, use_tc_tiling_on_sc=False),
        interpret=interpret,
        scratch_types=[
            pltpu.VMEM((P,), jnp.float32),
            pltpu.VMEM((P,), jnp.float32),
            pltpu.VMEM((P,), jnp.float32),
            pltpu.VMEM((P,), jnp.float32),
            pltpu.VMEM((4, P // G, G), jnp.int32),
            pltpu.VMEM((P,), jnp.float32),
            pltpu.VMEM((P,), jnp.float32),
            pltpu.VMEM((P,), jnp.float32),
            pltpu.VMEM((P,), jnp.float32),
            pltpu.VMEM((4 * P, C), jnp.float32),
            pltpu.VMEM((P, C), jnp.float32),
            pltpu.VMEM((LANES,), jnp.float32),
            pltpu.SemaphoreType.DMA,
            pltpu.SemaphoreType.DMA,
            pltpu.SemaphoreType.DMA,
            pltpu.SemaphoreType.DMA,
        ],
    )
    def sc_call(cpb_hbm, cum_hbm, img_hbm, wk_hbm, out_hbm,
                glx, gly, ckx, cky, idx4, w_a, w_b, w_c, w_d,
                rows4, acc, wkv, sem0, sem1, sem2, sem3):
        wbufs = (w_a, w_b, w_c, w_d)
        sems = (sem0, sem1, sem2, sem3)
        cid = lax.axis_index("c")
        sid = lax.axis_index("s")
        f = sid * NC + cid
        b = f // L
        l = f % L
        k0 = jnp.maximum(l - (WINDOW - 1), 0)
        pltpu.sync_copy(wk_hbm, wkv)

        def chunk_body(ci, _):
            c0 = pl.multiple_of(ci * P, P)
            pltpu.sync_copy(cpb_hbm.at[f, 0, pl.ds(c0, P)], glx)
            pltpu.sync_copy(cpb_hbm.at[f, 1, pl.ds(c0, P)], gly)
            pltpu.sync_copy(img_hbm.at[pl.ds(f * HW + c0, P), :], acc)

            def k_body(k, _):
                bk = b * L + k
                pltpu.sync_copy(cum_hbm.at[bk, 0, pl.ds(c0, P)], ckx)
                pltpu.sync_copy(cum_hbm.at[bk, 1, pl.ds(c0, P)], cky)
                wkd = _splat(wkv, l - k)
                rowbase = bk * HW

                for g in range(P // LANES):
                    s = pl.ds(g * LANES, LANES)
                    gx = glx[s] - ckx[s]
                    gy = gly[s] - cky[s]
                    ix = (gx + 1.0) * (W * 0.5) - 0.5
                    iy = (gy + 1.0) * (H * 0.5) - 0.5
                    xt = ix.astype(jnp.int32)
                    x0 = xt - (xt.astype(jnp.float32) > ix).astype(jnp.int32)
                    fx = ix - x0.astype(jnp.float32)
                    yt = iy.astype(jnp.int32)
                    y0 = yt - (yt.astype(jnp.float32) > iy).astype(jnp.int32)
                    fy = iy - y0.astype(jnp.float32)
                    x1 = x0 + 1
                    y1 = y0 + 1
                    vx0 = (x0 >= 0) & (x0 < W)
                    vx1 = (x1 >= 0) & (x1 < W)
                    vy0 = (y0 >= 0) & (y0 < H)
                    vy1 = (y1 >= 0) & (y1 < H)
                    cx0 = jnp.clip(x0, 0, W - 1)
                    cx1 = jnp.clip(x1, 0, W - 1)
                    ry0 = rowbase + jnp.clip(y0, 0, H - 1) * W
                    ry1 = rowbase + jnp.clip(y1, 0, H - 1) * W
                    ofx = 1.0 - fx
                    ofy = 1.0 - fy
                    zero = jnp.zeros((LANES,), jnp.float32)
                    wa = jnp.where(vx0 & vy0, wkd * (ofx * ofy), zero)
                    wb = jnp.where(vx0 & vy1, wkd * (ofx * fy), zero)
                    wc = jnp.where(vx1 & vy0, wkd * (fx * ofy), zero)
                    wd = jnp.where(vx1 & vy1, wkd * (fx * fy), zero)
                    j, o = divmod(g * LANES, G)
                    so = pl.ds(o, LANES)
                    idx4[0, j, so] = ry0 + cx0
                    idx4[1, j, so] = ry1 + cx0
                    idx4[2, j, so] = ry0 + cx1
                    idx4[3, j, so] = ry1 + cx1
                    w_a[s] = wa
                    w_b[s] = wb
                    w_c[s] = wc
                    w_d[s] = wd

                cps = [
                    [
                        pltpu.async_copy(
                            img_hbm.at[idx4.at[t, j]],
                            rows4.at[pl.ds(t * P + j * G, G), :],
                            sems[t],
                        )
                        for j in range(P // G)
                    ]
                    for t in range(4)
                ]
                for t in range(4):
                    for cp in cps[t]:
                        cp.wait()

                    def accum_body(p, _, t=t):
                        wv = _splat(wbufs[t], p)
                        plsc.addupdate(
                            acc.at[p, pl.ds(0, LANES)],
                            wv * rows4[t * P + p, pl.ds(0, LANES)])
                        plsc.addupdate(
                            acc.at[p, pl.ds(LANES, LANES)],
                            wv * rows4[t * P + p, pl.ds(LANES, LANES)])
                        return 0
                    lax.fori_loop(0, P, accum_body, 0)
                return 0

            lax.fori_loop(k0, l, k_body, 0)
            pltpu.sync_copy(acc, out_hbm.at[f, pl.ds(c0, P), :])
            return 0

        lax.fori_loop(0, HW // P, chunk_body, 0)

    return sc_call


def kernel(flows, images, decay_log):
    B, L, C, H, W = images.shape
    HW = H * W
    cum = jnp.cumsum(flows.astype(jnp.float32), axis=1)
    gx = jnp.linspace(-1.0 + 1.0 / W, 1.0 - 1.0 / W, W)
    gy = jnp.linspace(-1.0 + 1.0 / H, 1.0 - 1.0 / H, H)
    mx, my = jnp.meshgrid(gx, gy, indexing="xy")
    base = jnp.stack([mx, my], axis=0).astype(jnp.float32)
    cpb = cum + base[None, None]
    cum2 = cum.reshape(B * L, 2, HW)
    cpb2 = cpb.reshape(B * L, 2, HW)
    imgflat = (images.astype(jnp.float32)
               .transpose(0, 1, 3, 4, 2)
               .reshape(B * L * HW, C))
    decay = jnp.exp(decay_log)
    dist = jnp.arange(LANES, dtype=jnp.float32)
    wks = jnp.exp(-decay * dist)
    out = _make_sc_call(B, L, C, H, W)(cpb2, cum2, imgflat, wks)
    out = out.reshape(B, L, H, W, C).transpose(0, 1, 4, 2, 3)
    return out.astype(images.dtype)


# speedup vs baseline: 16.7986x; 2.3871x over previous
import functools

import jax
import jax.numpy as jnp
from jax import lax
from jax.experimental import pallas as pl
from jax.experimental.pallas import tpu as pltpu
from jax.experimental.pallas import tpu_sc as plsc

WINDOW = 8
NC = 2
NS = 16
LANES = 16


def _splat(ref, i):
    return plsc.load_gather(ref, [jnp.full((LANES,), i, jnp.int32)])


def _make_sc_call(B, L, C, H, W):
    HW = H * W
    P = min(512, HW)
    G = min(128, P)
    CAP = 4 * P + LANES
    NSTREAM = (4 * P) // G
    mesh = plsc.VectorSubcoreMesh(core_axis_name="c", subcore_axis_name="s",
                                  num_cores=NC, num_subcores=NS)

    @functools.partial(
        pl.kernel,
        out_type=jax.ShapeDtypeStruct((B * L, HW, C), jnp.float32),
        mesh=mesh,
        compiler_params=pltpu.CompilerParams(
            needs_layout_passes=False, use_tc_tiling_on_sc=False),
        scratch_types=[
            pltpu.VMEM((P,), jnp.float32),
            pltpu.VMEM((P,), jnp.float32),
            pltpu.VMEM((P,), jnp.float32),
            pltpu.VMEM((P,), jnp.float32),
            pltpu.VMEM((CAP,), jnp.int32),
            pltpu.VMEM((CAP,), jnp.float32),
            pltpu.VMEM((CAP,), jnp.int32),
            pltpu.VMEM((4 * P, C), jnp.float32),
            pltpu.VMEM((P, C), jnp.float32),
            pltpu.VMEM((LANES,), jnp.float32),
            pltpu.SemaphoreType.DMA,
        ],
    )
    def sc_call(cpb_hbm, cum_hbm, img_hbm, wk_hbm, out_hbm,
                glx, gly, ckx, cky, idxb, wcb, pcb, rowsb, acc, wkv, sem):
        cid = lax.axis_index("c")
        sid = lax.axis_index("s")
        f = sid * NC + cid
        b = f // L
        l = f % L
        k0 = jnp.maximum(l - (WINDOW - 1), 0)
        pltpu.sync_copy(wk_hbm, wkv)

        def zidx_body(z, _):
            z0 = pl.multiple_of(z * LANES, LANES)
            idxb[pl.ds(z0, LANES)] = jnp.zeros((LANES,), jnp.int32)
            return 0
        lax.fori_loop(0, CAP // LANES, zidx_body, 0)

        def chunk_body(ci, _):
            c0 = pl.multiple_of(ci * P, P)
            pltpu.sync_copy(cpb_hbm.at[f, 0, pl.ds(c0, P)], glx)
            pltpu.sync_copy(cpb_hbm.at[f, 1, pl.ds(c0, P)], gly)
            pltpu.sync_copy(img_hbm.at[pl.ds(f * HW + c0, P), :], acc)

            def k_body(k, _):
                bk = b * L + k
                pltpu.sync_copy(cum_hbm.at[bk, 0, pl.ds(c0, P)], ckx)
                pltpu.sync_copy(cum_hbm.at[bk, 1, pl.ds(c0, P)], cky)
                wkd = _splat(wkv, l - k)
                rowbase = bk * HW
                n = jnp.int32(0)

                for g in range(P // LANES):
                    s = pl.ds(g * LANES, LANES)
                    gx = glx[s] - ckx[s]
                    gy = gly[s] - cky[s]
                    ix = (gx + 1.0) * (W * 0.5) - 0.5
                    iy = (gy + 1.0) * (H * 0.5) - 0.5
                    xt = ix.astype(jnp.int32)
                    x0 = xt - (xt.astype(jnp.float32) > ix).astype(jnp.int32)
                    fx = ix - x0.astype(jnp.float32)
                    yt = iy.astype(jnp.int32)
                    y0 = yt - (yt.astype(jnp.float32) > iy).astype(jnp.int32)
                    fy = iy - y0.astype(jnp.float32)
                    x1 = x0 + 1
                    y1 = y0 + 1
                    vx0 = (x0 >= 0) & (x0 < W)
                    vx1 = (x1 >= 0) & (x1 < W)
                    vy0 = (y0 >= 0) & (y0 < H)
                    vy1 = (y1 >= 0) & (y1 < H)
                    ofx = 1.0 - fx
                    ofy = 1.0 - fy
                    piota = lax.iota(jnp.int32, LANES) + g * LANES
                    rx0 = rowbase + y0 * W + x0
                    taps = (
                        (vx0 & vy0, wkd * (ofx * ofy), rx0),
                        (vx0 & vy1, wkd * (ofx * fy), rx0 + W),
                        (vx1 & vy0, wkd * (fx * ofy), rx0 + 1),
                        (vx1 & vy1, wkd * (fx * fy), rx0 + W + 1),
                    )
                    for mask, wt, it in taps:
                        plsc.store_compressed(idxb.at[pl.ds(n, LANES)],
                                              it, mask=mask)
                        plsc.store_compressed(wcb.at[pl.ds(n, LANES)],
                                              wt, mask=mask)
                        plsc.store_compressed(pcb.at[pl.ds(n, LANES)],
                                              piota, mask=mask)
                        n = n + jnp.sum(mask.astype(jnp.int32), axis=0)

                cps = []
                for j in range(NSTREAM):
                    @pl.when(j * G < n)
                    def _(j=j):
                        cps.append(pltpu.async_copy(
                            img_hbm.at[idxb.at[pl.ds(j * G, G)]],
                            rowsb.at[pl.ds(j * G, G), :], sem))
                for j in range(NSTREAM):
                    @pl.when(j * G < n)
                    def _(j=j):
                        cps[0].wait()
                        del cps[0]

                def accum_body(i, _):
                    wv = _splat(wcb, i)
                    p = pcb[pl.ds(i, LANES)][0]
                    plsc.addupdate(acc.at[p, pl.ds(0, LANES)],
                                   wv * rowsb[i, pl.ds(0, LANES)])
                    plsc.addupdate(acc.at[p, pl.ds(LANES, LANES)],
                                   wv * rowsb[i, pl.ds(LANES, LANES)])
                    return 0
                lax.fori_loop(0, n, accum_body, 0)
                return 0

            lax.fori_loop(k0, l, k_body, 0)
            pltpu.sync_copy(acc, out_hbm.at[f, pl.ds(c0, P), :])
            return 0

        lax.fori_loop(0, HW // P, chunk_body, 0)

    return sc_call


def kernel(flows, images, decay_log):
    B, L, C, H, W = images.shape
    HW = H * W
    cum = jnp.cumsum(flows.astype(jnp.float32), axis=1)
    gx = jnp.linspace(-1.0 + 1.0 / W, 1.0 - 1.0 / W, W)
    gy = jnp.linspace(-1.0 + 1.0 / H, 1.0 - 1.0 / H, H)
    mx, my = jnp.meshgrid(gx, gy, indexing="xy")
    base = jnp.stack([mx, my], axis=0).astype(jnp.float32)
    cpb = cum + base[None, None]
    cum2 = cum.reshape(B * L, 2, HW)
    cpb2 = cpb.reshape(B * L, 2, HW)
    imgflat = (images.astype(jnp.float32)
               .transpose(0, 1, 3, 4, 2)
               .reshape(B * L * HW, C))
    decay = jnp.exp(decay_log)
    dist = jnp.arange(LANES, dtype=jnp.float32)
    wks = jnp.exp(-decay * dist)
    out = _make_sc_call(B, L, C, H, W)(cpb2, cum2, imgflat, wks)
    out = out.reshape(B, L, H, W, C).transpose(0, 1, 4, 2, 3)
    return out.astype(images.dtype)
